# Initial kernel scaffold; baseline (speedup 1.0000x reference)
import jax
import jax.numpy as jnp
from jax.experimental import pallas as pl


def kernel(mu, x, edge_index, W1, W2):
    raise NotImplementedError("write your pallas kernel here")


# speedup vs baseline: 3.4051x; 3.4051x over previous
import functools

import jax
import jax.numpy as jnp
from jax import lax
from jax.experimental import pallas as pl
from jax.experimental.pallas import tpu as pltpu
from jax.experimental.pallas import tpu_sc as plsc

N_NODES = 10000
N_EDGES = 320000
D = 128
VD = 24

NC = 2
NS = 16
NW = NC * NS
CHUNK = 128
EPW = 10240
NCHUNK = EPW // CHUNK
E_PAD = EPW * NW
ACC_ROWS = 10240
ZROWS = ACC_ROWS // NS
OUT_RPS = 624
TRASH_ROW = N_NODES

_mesh = plsc.VectorSubcoreMesh(core_axis_name="c", subcore_axis_name="s")


@functools.partial(
    pl.kernel,
    out_type=jax.ShapeDtypeStruct((NC, N_NODES, D), jnp.float32),
    mesh=_mesh,
    scratch_types=[
        pltpu.VMEM_SHARED((ACC_ROWS, D), jnp.float32),
        pltpu.VMEM((CHUNK,), jnp.int32),
        pltpu.VMEM((CHUNK,), jnp.int32),
        pltpu.VMEM((CHUNK, D), jnp.float32),
        pltpu.SemaphoreType.DMA,
    ],
)
def _segsum_sc(mu_hbm, src_hbm, dst_hbm, zeros_hbm, out_hbm,
               acc, idx_v, dst_v, rows_v, sem):
    c = lax.axis_index("c")
    s = lax.axis_index("s")
    wid = s * NC + c

    pltpu.sync_copy(zeros_hbm, acc.at[pl.ds(s * ZROWS, ZROWS)])
    plsc.subcore_barrier()

    base = wid * EPW

    def body(k, carry):
        off = base + k * CHUNK
        pltpu.sync_copy(src_hbm.at[pl.ds(off, CHUNK)], idx_v)
        pltpu.sync_copy(dst_hbm.at[pl.ds(off, CHUNK)], dst_v)
        pltpu.async_copy(mu_hbm.at[idx_v], rows_v, sem).wait()
        pltpu.sync_copy(rows_v, acc.at[dst_v], add=True)
        return carry

    lax.fori_loop(0, NCHUNK, body, 0)
    plsc.subcore_barrier()

    @pl.when(s < NS - 1)
    def _copy_main():
        pltpu.sync_copy(acc.at[pl.ds(s * OUT_RPS, OUT_RPS)],
                        out_hbm.at[c, pl.ds(s * OUT_RPS, OUT_RPS)])

    @pl.when(s == NS - 1)
    def _copy_tail():
        tail = N_NODES - (NS - 1) * OUT_RPS
        pltpu.sync_copy(acc.at[pl.ds((NS - 1) * OUT_RPS, tail)],
                        out_hbm.at[c, pl.ds((NS - 1) * OUT_RPS, tail)])


def _dense_body(x_ref, w1t_ref, p0_ref, p1_ref, w2t_ref, o_ref):
    xh = jnp.dot(x_ref[...], w1t_ref[...], preferred_element_type=jnp.float32)
    agg = jnp.dot(p0_ref[...] + p1_ref[...], w2t_ref[...],
                  preferred_element_type=jnp.float32)
    o_ref[...] = jnp.maximum(xh + agg, 0.0)


_ROWS_BLK = 1000

_dense = pl.pallas_call(
    _dense_body,
    grid=(N_NODES // _ROWS_BLK,),
    in_specs=[
        pl.BlockSpec((_ROWS_BLK, VD), lambda i: (i, 0)),
        pl.BlockSpec((VD, D), lambda i: (0, 0)),
        pl.BlockSpec((_ROWS_BLK, D), lambda i: (i, 0)),
        pl.BlockSpec((_ROWS_BLK, D), lambda i: (i, 0)),
        pl.BlockSpec((D, D), lambda i: (0, 0)),
    ],
    out_specs=pl.BlockSpec((_ROWS_BLK, D), lambda i: (i, 0)),
    out_shape=jax.ShapeDtypeStruct((N_NODES, D), jnp.float32),
)


def kernel(mu, x, edge_index, W1, W2):
    ei = edge_index.astype(jnp.int32)
    pad = E_PAD - N_EDGES
    src_p = jnp.concatenate([ei[1], jnp.zeros((pad,), jnp.int32)])
    dst_p = jnp.concatenate([ei[0], jnp.full((pad,), TRASH_ROW, jnp.int32)])
    zeros = jnp.zeros((ZROWS, D), jnp.float32)
    partials = _segsum_sc(mu, src_p, dst_p, zeros)
    return _dense(x, W1.T, partials[0], partials[1], W2.T)


# speedup vs baseline: 4.4664x; 1.3117x over previous
import functools

import jax
import jax.numpy as jnp
from jax import lax
from jax.experimental import pallas as pl
from jax.experimental.pallas import tpu as pltpu
from jax.experimental.pallas import tpu_sc as plsc

N_NODES = 10000
N_EDGES = 320000
D = 128
VD = 24

NC = 2
NS = 16
NW = NC * NS
CHUNK = 128
EPW = 10240
NCHUNK = EPW // CHUNK
E_PAD = EPW * NW
ACC_ROWS = 10112
ZROWS = ACC_ROWS // NS
OUT_RPS = 624
TRASH_ROW = N_NODES

_mesh = plsc.VectorSubcoreMesh(core_axis_name="c", subcore_axis_name="s")


NBUF = 2
SUP = 8
NSUP = NCHUNK // SUP


@functools.partial(
    pl.kernel,
    out_type=jax.ShapeDtypeStruct((NC, N_NODES, D), jnp.float32),
    mesh=_mesh,
    scratch_types=[
        pltpu.VMEM_SHARED((ACC_ROWS, D), jnp.float32),
        pltpu.VMEM((2, SUP, CHUNK), jnp.int32),
        pltpu.VMEM((2, SUP, CHUNK), jnp.int32),
        pltpu.VMEM((NBUF, CHUNK, D), jnp.float32),
        pltpu.SemaphoreType.DMA((2,)),
        pltpu.SemaphoreType.DMA((2,)),
        pltpu.SemaphoreType.DMA((NBUF,)),
        pltpu.SemaphoreType.DMA((NBUF,)),
    ],
)
def _segsum_sc(mu_hbm, src_hbm, dst_hbm, zeros_hbm, out_hbm,
               acc, src_v, dst_v, rows_v, isem_s, isem_d, gsem, ssem):
    c = lax.axis_index("c")
    s = lax.axis_index("s")
    wid = s * NC + c

    def start_idx(u, m):
        pltpu.async_copy(src_hbm.at[wid, u], src_v.at[m], isem_s.at[m])
        pltpu.async_copy(dst_hbm.at[wid, u], dst_v.at[m], isem_d.at[m])

    def wait_idx(u, m):
        pltpu.make_async_copy(src_hbm.at[wid, u], src_v.at[m],
                              isem_s.at[m]).wait()
        pltpu.make_async_copy(dst_hbm.at[wid, u], dst_v.at[m],
                              isem_d.at[m]).wait()

    def start_gather(k, b):
        pltpu.async_copy(mu_hbm.at[src_v.at[(k // SUP) % 2, k % SUP]],
                         rows_v.at[b], gsem.at[b])

    def wait_gather(k, b):
        pltpu.make_async_copy(mu_hbm.at[src_v.at[(k // SUP) % 2, k % SUP]],
                              rows_v.at[b], gsem.at[b]).wait()

    def start_scatter(k, b):
        pltpu.async_copy(rows_v.at[b], acc.at[dst_v.at[(k // SUP) % 2, k % SUP]],
                         ssem.at[b], add=True)

    def wait_scatter(k, b):
        pltpu.make_async_copy(rows_v.at[b],
                              acc.at[dst_v.at[(k // SUP) % 2, k % SUP]],
                              ssem.at[b]).wait()

    start_idx(0, 0)
    start_idx(1, 1)
    pltpu.sync_copy(zeros_hbm, acc.at[pl.ds(s * ZROWS, ZROWS)])
    plsc.subcore_barrier()
    wait_idx(0, 0)
    start_gather(0, 0)
    start_gather(1, 1)

    def super_body(u, carry):
        wait_idx(u + 1, (u + 1) % 2)
        k0 = u * SUP
        for j in range(SUP):
            b = j % NBUF
            wait_gather(k0 + j, b)
            start_scatter(k0 + j, b)
            wait_scatter(k0 + j, b)
            start_gather(k0 + j + NBUF, b)

        @pl.when(u <= NSUP - 3)
        def _():
            start_idx(u + 2, u % 2)

        return carry

    lax.fori_loop(0, NSUP - 1, super_body, 0)

    k0 = (NSUP - 1) * SUP
    for j in range(SUP):
        b = j % NBUF
        wait_gather(k0 + j, b)
        start_scatter(k0 + j, b)
        wait_scatter(k0 + j, b)
        if k0 + j + NBUF < NCHUNK:
            start_gather(k0 + j + NBUF, b)

    plsc.subcore_barrier()

    @pl.when(s < NS - 1)
    def _copy_main():
        pltpu.sync_copy(acc.at[pl.ds(s * OUT_RPS, OUT_RPS)],
                        out_hbm.at[c, pl.ds(s * OUT_RPS, OUT_RPS)])

    @pl.when(s == NS - 1)
    def _copy_tail():
        tail = N_NODES - (NS - 1) * OUT_RPS
        pltpu.sync_copy(acc.at[pl.ds((NS - 1) * OUT_RPS, tail)],
                        out_hbm.at[c, pl.ds((NS - 1) * OUT_RPS, tail)])


def _dense_body(x_ref, w1t_ref, p0_ref, p1_ref, w2t_ref, o_ref):
    xh = jnp.dot(x_ref[...], w1t_ref[...], preferred_element_type=jnp.float32)
    agg = jnp.dot(p0_ref[...] + p1_ref[...], w2t_ref[...],
                  preferred_element_type=jnp.float32)
    o_ref[...] = jnp.maximum(xh + agg, 0.0)


_ROWS_BLK = 1000

_dense = pl.pallas_call(
    _dense_body,
    grid=(N_NODES // _ROWS_BLK,),
    in_specs=[
        pl.BlockSpec((_ROWS_BLK, VD), lambda i: (i, 0)),
        pl.BlockSpec((VD, D), lambda i: (0, 0)),
        pl.BlockSpec((_ROWS_BLK, D), lambda i: (i, 0)),
        pl.BlockSpec((_ROWS_BLK, D), lambda i: (i, 0)),
        pl.BlockSpec((D, D), lambda i: (0, 0)),
    ],
    out_specs=pl.BlockSpec((_ROWS_BLK, D), lambda i: (i, 0)),
    out_shape=jax.ShapeDtypeStruct((N_NODES, D), jnp.float32),
)


def kernel(mu, x, edge_index, W1, W2):
    ei = edge_index.astype(jnp.int32)
    pad = E_PAD - N_EDGES
    src_p = jnp.concatenate([ei[1], jnp.zeros((pad,), jnp.int32)])
    src_p = src_p.reshape(NW, NSUP, SUP, CHUNK)
    dst_p = jnp.concatenate([ei[0], jnp.full((pad,), TRASH_ROW, jnp.int32)])
    dst_p = dst_p.reshape(NW, NSUP, SUP, CHUNK)
    zeros = jnp.zeros((ZROWS, D), jnp.float32)
    partials = _segsum_sc(mu, src_p, dst_p, zeros)
    return _dense(x, W1.T, partials[0], partials[1], W2.T)
